# Initial kernel scaffold; baseline (speedup 1.0000x reference)
import jax
import jax.numpy as jnp
from jax.experimental import pallas as pl


def kernel(graph, state, next_state, W1, b1, W2, b2, W3, b3):
    raise NotImplementedError("write your pallas kernel here")


# speedup vs baseline: 1.4844x; 1.4844x over previous
import functools

import jax
import jax.numpy as jnp
from jax.experimental import pallas as pl
from jax.experimental.pallas import tpu as pltpu

B = 16384
OBS = 128
GENC = 64
HID = 128
SKILL = 64
NF = 8
INP = GENC + OBS + OBS

TILE = 512


def _dense_body(g_ref, s_ref, n_ref, w1_ref, b1_ref, w2_ref, b2_ref,
                w3_ref, b3_ref, out_ref):
    g = g_ref[...]
    x = jnp.concatenate([g, s_ref[...], n_ref[...]], axis=1)
    acc = jnp.zeros((g.shape[0], SKILL), dtype=jnp.float32)
    for i in range(NF):
        h = jnp.maximum(
            jnp.dot(x, w1_ref[i], preferred_element_type=jnp.float32)
            + b1_ref[i][None, :], 0.0)
        h = jnp.maximum(
            jnp.dot(h, w2_ref[i], preferred_element_type=jnp.float32)
            + b2_ref[i][None, :], 0.0)
        o = (jnp.dot(h, w3_ref[i], preferred_element_type=jnp.float32)
             + b3_ref[i][None, :])
        acc = jnp.where(g[:, i:i + 1] == 1.0, o, acc)
    out_ref[...] = acc


def kernel(graph, state, next_state, W1, b1, W2, b2, W3, b3):
    grid = (B // TILE,)
    return pl.pallas_call(
        _dense_body,
        grid=grid,
        in_specs=[
            pl.BlockSpec((TILE, GENC), lambda t: (t, 0)),
            pl.BlockSpec((TILE, OBS), lambda t: (t, 0)),
            pl.BlockSpec((TILE, OBS), lambda t: (t, 0)),
            pl.BlockSpec((NF, INP, HID), lambda t: (0, 0, 0)),
            pl.BlockSpec((NF, HID), lambda t: (0, 0)),
            pl.BlockSpec((NF, HID, HID), lambda t: (0, 0, 0)),
            pl.BlockSpec((NF, HID), lambda t: (0, 0)),
            pl.BlockSpec((NF, HID, SKILL), lambda t: (0, 0, 0)),
            pl.BlockSpec((NF, SKILL), lambda t: (0, 0)),
        ],
        out_specs=pl.BlockSpec((TILE, SKILL), lambda t: (t, 0)),
        out_shape=jax.ShapeDtypeStruct((B, SKILL), jnp.float32),
    )(graph, state, next_state, W1, b1, W2, b2, W3, b3)
